# baseline (device time: 11484 ns/iter reference)
import jax
import jax.numpy as jnp
from jax import lax
from jax.experimental import pallas as pl
from jax.experimental.pallas import tpu as pltpu

CHUNK_ROWS = (64, 64, 64, 56, 8)
N_CHUNKS = len(CHUNK_ROWS)
CHUNK_OFFS = tuple(sum(CHUNK_ROWS[:k]) for k in range(N_CHUNKS))


def kernel(x):
    _, m, n_per = x.shape
    n_dev_y = 2
    assert sum(CHUNK_ROWS) == m

    def body(x_ref, out_ref, xrecv_ref,
             xsend_sems, xrecv_sems, ysend_sems, yrecv_sems):
        my_x = lax.axis_index("x")
        my_y = lax.axis_index("y")
        x_partner = (1 - my_x, my_y)
        y_partner = (my_x, 1 - my_y)

        barrier_sem = pltpu.get_barrier_semaphore()
        for nbr in (x_partner, y_partner):
            pl.semaphore_signal(
                barrier_sem, inc=1,
                device_id=nbr, device_id_type=pl.DeviceIdType.MESH,
            )
        pl.semaphore_wait(barrier_sem, 2)

        rows = lambda k: pl.ds(CHUNK_OFFS[k], CHUNK_ROWS[k])

        rdma_x = []
        for k in range(N_CHUNKS):
            r = pltpu.make_async_remote_copy(
                src_ref=x_ref.at[0, rows(k), :],
                dst_ref=xrecv_ref.at[rows(k), :],
                send_sem=xsend_sems.at[k],
                recv_sem=xrecv_sems.at[k],
                device_id=x_partner,
                device_id_type=pl.DeviceIdType.MESH,
            )
            r.start()
            rdma_x.append(r)

        own_col = pl.ds(my_y * n_per, n_per)
        rdma_y = []
        for k in range(N_CHUNKS):
            rdma_x[k].wait_recv()
            out_ref[rows(k), own_col] = (
                x_ref[0, rows(k), :] + xrecv_ref[rows(k), :]
            )
            r = pltpu.make_async_remote_copy(
                src_ref=out_ref.at[rows(k), own_col],
                dst_ref=out_ref.at[rows(k), own_col],
                send_sem=ysend_sems.at[k],
                recv_sem=yrecv_sems.at[k],
                device_id=y_partner,
                device_id_type=pl.DeviceIdType.MESH,
            )
            r.start()
            rdma_y.append(r)

        for k in range(N_CHUNKS):
            rdma_y[k].wait_recv()

        for k in range(N_CHUNKS):
            rdma_x[k].wait_send()
            rdma_y[k].wait_send()

    return pl.pallas_call(
        body,
        out_shape=jax.ShapeDtypeStruct((m, n_dev_y * n_per), x.dtype),
        in_specs=[pl.BlockSpec(memory_space=pltpu.VMEM)],
        out_specs=pl.BlockSpec(memory_space=pltpu.VMEM),
        scratch_shapes=[
            pltpu.VMEM((m, n_per), x.dtype),
            pltpu.SemaphoreType.DMA((N_CHUNKS,)),
            pltpu.SemaphoreType.DMA((N_CHUNKS,)),
            pltpu.SemaphoreType.DMA((N_CHUNKS,)),
            pltpu.SemaphoreType.DMA((N_CHUNKS,)),
        ],
        compiler_params=pltpu.CompilerParams(collective_id=0),
    )(x)


# device time: 10935 ns/iter; 1.0502x vs baseline; 1.0502x over previous
import jax
import jax.numpy as jnp
from jax import lax
from jax.experimental import pallas as pl
from jax.experimental.pallas import tpu as pltpu

N_CHUNKS = 8


def kernel(x):
    _, m, n_per = x.shape
    n_dev_y = 2
    mc = m // N_CHUNKS

    def body(x_ref, out_ref, xvmem_ref, xrecv_ref,
             in_sem, in_sem2, xsend_sems, xrecv_sems, ysend_sems, yrecv_sems):
        my_x = lax.axis_index("x")
        my_y = lax.axis_index("y")
        x_partner = (1 - my_x, my_y)
        y_partner = (my_x, 1 - my_y)

        half = m // 2
        in_dma0 = pltpu.make_async_copy(
            x_ref.at[0, pl.ds(0, half), :], xvmem_ref.at[pl.ds(0, half), :],
            in_sem,
        )
        in_dma1 = pltpu.make_async_copy(
            x_ref.at[0, pl.ds(half, half), :],
            xvmem_ref.at[pl.ds(half, half), :],
            in_sem2,
        )
        in_dma0.start()
        in_dma1.start()

        barrier_sem = pltpu.get_barrier_semaphore()
        for nbr in (x_partner, y_partner):
            pl.semaphore_signal(
                barrier_sem, inc=1,
                device_id=nbr, device_id_type=pl.DeviceIdType.MESH,
            )
        pl.semaphore_wait(barrier_sem, 2)
        in_dma0.wait()

        rows = lambda k: pl.ds(k * mc, mc)

        rdma_x = []
        for k in range(N_CHUNKS):
            if k == N_CHUNKS // 2:
                in_dma1.wait()
            r = pltpu.make_async_remote_copy(
                src_ref=xvmem_ref.at[rows(k), :],
                dst_ref=xrecv_ref.at[rows(k), :],
                send_sem=xsend_sems.at[k],
                recv_sem=xrecv_sems.at[k],
                device_id=x_partner,
                device_id_type=pl.DeviceIdType.MESH,
            )
            r.start()
            rdma_x.append(r)

        own_col = pl.ds(my_y * n_per, n_per)
        rdma_y = []
        for k in range(N_CHUNKS):
            rdma_x[k].wait_recv()
            out_ref[rows(k), own_col] = (
                xvmem_ref[rows(k), :] + xrecv_ref[rows(k), :]
            )
            r = pltpu.make_async_remote_copy(
                src_ref=out_ref.at[rows(k), own_col],
                dst_ref=out_ref.at[rows(k), own_col],
                send_sem=ysend_sems.at[k],
                recv_sem=yrecv_sems.at[k],
                device_id=y_partner,
                device_id_type=pl.DeviceIdType.MESH,
            )
            r.start()
            rdma_y.append(r)

        for k in range(N_CHUNKS):
            rdma_y[k].wait_recv()

        for k in range(N_CHUNKS):
            rdma_x[k].wait_send()
            rdma_y[k].wait_send()

    return pl.pallas_call(
        body,
        out_shape=jax.ShapeDtypeStruct((m, n_dev_y * n_per), x.dtype),
        in_specs=[pl.BlockSpec(memory_space=pl.ANY)],
        out_specs=pl.BlockSpec(memory_space=pltpu.VMEM),
        scratch_shapes=[
            pltpu.VMEM((m, n_per), x.dtype),
            pltpu.VMEM((m, n_per), x.dtype),
            pltpu.SemaphoreType.DMA,
            pltpu.SemaphoreType.DMA,
            pltpu.SemaphoreType.DMA((N_CHUNKS,)),
            pltpu.SemaphoreType.DMA((N_CHUNKS,)),
            pltpu.SemaphoreType.DMA((N_CHUNKS,)),
            pltpu.SemaphoreType.DMA((N_CHUNKS,)),
        ],
        compiler_params=pltpu.CompilerParams(collective_id=0),
    )(pltpu.with_memory_space_constraint(x, pltpu.MemorySpace.HBM))
